# baseline (speedup 1.0000x reference)
import functools

import jax
import jax.numpy as jnp
from jax import lax
from jax.experimental import pallas as pl
from jax.experimental.pallas import tpu as pltpu
from jax.experimental.pallas import tpu_sc as plsc

B, V = 1024, 100000
TOPK = 256
V_BLOCK = 5000
N_STEPS = V // V_BLOCK

_NC, _NS, _L = 2, 16, 16
_NW = _NC * _NS
_BPW = B // _NW


def _sc_pick_kernel(xt_hbm, yt_hbm, out_hbm, idx_v, rows_v, vals_v, sem):
    wid = lax.axis_index("s") * _NC + lax.axis_index("c")
    base = wid * _BPW
    pltpu.sync_copy(yt_hbm.at[pl.ds(base, _BPW)], idx_v)
    pltpu.async_copy(xt_hbm.at[idx_v], rows_v, sem).wait()
    for g in range(_BPW // _L):
        r_idx = lax.iota(jnp.int32, _L) + g * _L
        c_idx = r_idx + base
        vals_v[pl.ds(g * _L, _L)] = plsc.load_gather(rows_v, [r_idx, c_idx])
    pltpu.sync_copy(vals_v, out_hbm.at[pl.ds(base, _BPW)])


_sc_pick = functools.partial(
    pl.kernel,
    out_type=jax.ShapeDtypeStruct((B,), jnp.float32),
    mesh=plsc.VectorSubcoreMesh(core_axis_name="c", subcore_axis_name="s"),
    scratch_types=[
        pltpu.VMEM((_BPW,), jnp.int32),
        pltpu.VMEM((_BPW, B), jnp.float32),
        pltpu.VMEM((_BPW,), jnp.float32),
        pltpu.SemaphoreType.DMA,
    ],
    compiler_params=pltpu.CompilerParams(
        use_tc_tiling_on_sc=True, needs_layout_passes=False
    ),
)(_sc_pick_kernel)


def _s_accum_kernel(x_ref, s_ref):
    i = pl.program_id(0)

    @pl.when(i == 0)
    def _init():
        s_ref[...] = jnp.zeros_like(s_ref)

    s_ref[...] += jnp.sum(jnp.exp(x_ref[...]), axis=0, keepdims=True)


def _topk_mean_kernel(s_ref, p_ref, out_ref):
    ce = jnp.log(s_ref[...]) - p_ref[...]
    keys = jax.lax.bitcast_convert_type(ce, jnp.int32)

    def body(j, t):
        cand = t | (1 << (30 - j))
        cnt = jnp.sum((keys >= cand).astype(jnp.int32))
        return jnp.where(cnt >= TOPK, cand, t)

    t = jax.lax.fori_loop(0, 31, body, jnp.int32(0))
    t_val = jnp.max(jnp.where(keys == t, ce, -jnp.inf))
    gt = keys > t
    count_gt = jnp.sum(gt.astype(jnp.int32))
    sum_gt = jnp.sum(jnp.where(gt, ce, 0.0))
    loss = (sum_gt + (TOPK - count_gt).astype(jnp.float32) * t_val) / TOPK
    out_ref[...] = loss[None, None]


@jax.jit
def kernel(y_pred, y_true):
    xt = y_pred.T
    picked = _sc_pick(xt, y_true.astype(jnp.int32))
    s = pl.pallas_call(
        _s_accum_kernel,
        grid=(N_STEPS,),
        in_specs=[pl.BlockSpec((V_BLOCK, B), lambda i: (i, 0))],
        out_specs=pl.BlockSpec((1, B), lambda i: (0, 0)),
        out_shape=jax.ShapeDtypeStruct((1, B), jnp.float32),
        compiler_params=pltpu.CompilerParams(
            dimension_semantics=("arbitrary",),
        ),
    )(xt)
    loss = pl.pallas_call(
        _topk_mean_kernel,
        out_shape=jax.ShapeDtypeStruct((1, 1), jnp.float32),
    )(s, picked.reshape(1, B))
    return loss[0, 0]
